# Initial kernel scaffold; baseline (speedup 1.0000x reference)
import jax
import jax.numpy as jnp
from jax.experimental import pallas as pl


def kernel(category_ids, embedding_weight):
    raise NotImplementedError("write your pallas kernel here")


# speedup vs baseline: 1.8310x; 1.8310x over previous
import functools

import jax
import jax.numpy as jnp
from jax import lax
from jax.experimental import pallas as pl
from jax.experimental.pallas import tpu as pltpu
from jax.experimental.pallas import tpu_sc as plsc

NC = 2
NS = 16
NW = NC * NS
D = 64
CHUNK = 128
GROUP = 4


@functools.partial(jax.jit, static_argnames=("total",))
def _sc_gather(idx2d, table, total):
    bpw = total // NW
    nchunk = bpw // CHUNK
    ngroup = nchunk // GROUP
    gr = GROUP * CHUNK

    mesh = plsc.VectorSubcoreMesh(core_axis_name="c", subcore_axis_name="s")

    @functools.partial(
        pl.kernel,
        out_type=jax.ShapeDtypeStruct((total, D), jnp.float32),
        mesh=mesh,
        scratch_types=[
            pltpu.VMEM((nchunk, CHUNK), jnp.int32),
            pltpu.VMEM((gr, D), jnp.float32),
            pltpu.SemaphoreType.DMA,
        ],
        compiler_params=pltpu.CompilerParams(use_tc_tiling_on_sc=False),
    )
    def gather_kernel(idx_hbm, table_hbm, out_hbm, idx_v, rows_v, gsem):
        wid = lax.axis_index("s") * NC + lax.axis_index("c")
        base = wid * bpw
        pltpu.sync_copy(idx_hbm.at[pl.ds(wid * nchunk, nchunk)], idx_v)

        @pl.loop(0, ngroup)
        def _(g):
            copies = []
            for u in range(GROUP):
                copies.append(pltpu.async_copy(
                    table_hbm.at[idx_v.at[g * GROUP + u]],
                    rows_v.at[pl.ds(u * CHUNK, CHUNK)],
                    gsem,
                ))
            for c in copies:
                c.wait()
            pltpu.sync_copy(rows_v, out_hbm.at[pl.ds(base + g * gr, gr)])

    return gather_kernel(idx2d, table)


def kernel(category_ids, embedding_weight):
    b, h = category_ids.shape
    total = b * h
    idx2d = category_ids.reshape(total // CHUNK, CHUNK).astype(jnp.int32)
    out = _sc_gather(idx2d, embedding_weight, total)
    return out.reshape(b, h, D)


# speedup vs baseline: 1.8833x; 1.0286x over previous
import functools

import jax
import jax.numpy as jnp
from jax import lax
from jax.experimental import pallas as pl
from jax.experimental.pallas import tpu as pltpu
from jax.experimental.pallas import tpu_sc as plsc

NC = 2
NS = 16
NW = NC * NS
D = 64
CHUNK = 128
GROUP = 4


@functools.partial(jax.jit, static_argnames=("total",))
def _sc_gather(idx2d, table, total):
    bpw = total // NW
    nchunk = bpw // CHUNK
    ngroup = nchunk // GROUP
    gr = GROUP * CHUNK

    mesh = plsc.VectorSubcoreMesh(core_axis_name="c", subcore_axis_name="s")

    @functools.partial(
        pl.kernel,
        out_type=jax.ShapeDtypeStruct((total, D), jnp.float32),
        mesh=mesh,
        scratch_types=[
            pltpu.VMEM((nchunk, CHUNK), jnp.int32),
            pltpu.VMEM((2, gr, D), jnp.float32),
            pltpu.SemaphoreType.DMA,
            pltpu.SemaphoreType.DMA,
            pltpu.SemaphoreType.DMA,
            pltpu.SemaphoreType.DMA,
        ],
        compiler_params=pltpu.CompilerParams(use_tc_tiling_on_sc=False),
    )
    def gather_kernel(idx_hbm, table_hbm, out_hbm, idx_v, rows_v,
                      gsem0, gsem1, osem0, osem1):
        wid = lax.axis_index("s") * NC + lax.axis_index("c")
        base = wid * bpw
        gsem = (gsem0, gsem1)
        osem = (osem0, osem1)
        pltpu.sync_copy(idx_hbm.at[pl.ds(wid * nchunk, nchunk)], idx_v)

        def issue_gathers(g, s):
            for u in range(GROUP):
                pltpu.async_copy(
                    table_hbm.at[idx_v.at[g * GROUP + u]],
                    rows_v.at[s].at[pl.ds(u * CHUNK, CHUNK)],
                    gsem[s],
                )

        def drain_gathers(s):
            for u in range(GROUP):
                pltpu.make_async_copy(
                    table_hbm.at[idx_v.at[u]],
                    rows_v.at[s].at[pl.ds(u * CHUNK, CHUNK)],
                    gsem[s],
                ).wait()

        def drain_out(s):
            pltpu.make_async_copy(
                rows_v.at[s], out_hbm.at[pl.ds(base, gr)], osem[s],
            ).wait()

        issue_gathers(0, 0)

        @pl.loop(0, ngroup, step=2)
        def _(go):
            for s in range(2):
                g = go + s
                s2 = 1 - s
                drain_gathers(s)
                pltpu.async_copy(rows_v.at[s],
                                 out_hbm.at[pl.ds(base + g * gr, gr)],
                                 osem[s])
                if s == 0:
                    @pl.when(go > 0)
                    def _():
                        drain_out(s2)
                    issue_gathers(g + 1, s2)
                else:
                    drain_out(s2)

                    @pl.when(go + 2 < ngroup)
                    def _():
                        issue_gathers(g + 1, s2)

        drain_out(1)

    return gather_kernel(idx2d, table)


def kernel(category_ids, embedding_weight):
    b, h = category_ids.shape
    total = b * h
    idx2d = category_ids.reshape(total // CHUNK, CHUNK).astype(jnp.int32)
    out = _sc_gather(idx2d, embedding_weight, total)
    return out.reshape(b, h, D)
